# Initial kernel scaffold; baseline (speedup 1.0000x reference)
import jax
import jax.numpy as jnp
from jax.experimental import pallas as pl


def kernel(x, edge_index, edge_weight, edge_weight_multiplier, neuron_activation_threshold, W_fc, b_fc, sel_idx):
    raise NotImplementedError("write your pallas kernel here")


# speedup vs baseline: 15.5134x; 15.5134x over previous
import functools

import jax
import jax.numpy as jnp
from jax import lax
from jax.experimental import pallas as pl
from jax.experimental.pallas import tpu as pltpu
from jax.experimental.pallas import tpu_sc as plsc

_NUM_PASSES = 3
_NC = 2
_NS = 16
_LANES = 16


def _pick_chunk(epw: int) -> int:
    for c in range(4096, 15, -16):
        if epw % c == 0 and c % 8 == 0:
            return c
    return _LANES


def _make_propagate(B: int, N: int, E: int, interpret: bool = False):
    NW = _NC * _NS
    R = NW // B
    EPW = E // R
    C = _pick_chunk(EPW)
    n_chunks = EPW // C

    mesh = plsc.VectorSubcoreMesh(
        core_axis_name="c", subcore_axis_name="s",
        num_cores=_NC, num_subcores=_NS)

    @functools.partial(
        pl.kernel,
        out_type=jax.ShapeDtypeStruct((NW, N), jnp.float32),
        mesh=mesh,
        scratch_types=[
            pltpu.VMEM((N,), jnp.float32),
            pltpu.VMEM((N,), jnp.float32),
            pltpu.VMEM((C,), jnp.int32),
            pltpu.VMEM((C,), jnp.int32),
            pltpu.VMEM((C,), jnp.float32),
        ],
        compiler_params=pltpu.CompilerParams(needs_layout_passes=False),
        interpret=interpret,
    )
    def prop(h_hbm, src_hbm, dst_hbm, w_hbm, out_hbm,
             h_v, aggr_v, src_v, dst_v, w_v):
        cid = lax.axis_index("c")
        sid = lax.axis_index("s")
        wid = sid * _NC + cid
        b = wid % B
        r = wid // B

        pltpu.sync_copy(h_hbm.at[b], h_v)

        def zero_body(i, carry):
            aggr_v[pl.ds(i * _LANES, _LANES)] = jnp.zeros((_LANES,), jnp.float32)
            return carry
        lax.fori_loop(0, N // _LANES, zero_body, 0, unroll=8)

        base = r * EPW

        def chunk_body(j, carry):
            off = base + j * C
            pltpu.sync_copy(src_hbm.at[pl.ds(off, C)], src_v)
            pltpu.sync_copy(dst_hbm.at[pl.ds(off, C)], dst_v)
            pltpu.sync_copy(w_hbm.at[pl.ds(off, C)], w_v)

            def inner(i, icarry):
                sl = pl.ds(i * _LANES, _LANES)
                si = src_v[sl]
                di = dst_v[sl]
                wi = w_v[sl]
                vals = plsc.load_gather(h_v, [si])
                plsc.addupdate_scatter(aggr_v, [di], vals * wi)
                return icarry
            lax.fori_loop(0, C // _LANES, inner, carry, unroll=4)
            return carry
        lax.fori_loop(0, n_chunks, chunk_body, 0)

        pltpu.sync_copy(aggr_v, out_hbm.at[wid])

    return prop


def _effw_kernel(ew_ref, mult_ref, out_ref):
    out_ref[...] = ew_ref[...] * jax.nn.sigmoid(mult_ref[...])


def _compute_effw(edge_weight, edge_weight_multiplier, interpret=False):
    E = edge_weight.shape[0]
    cols = 512
    rows = E // cols
    ew2 = edge_weight.reshape(rows, cols)
    m2 = edge_weight_multiplier.reshape(rows, cols)
    out = pl.pallas_call(
        _effw_kernel,
        out_shape=jax.ShapeDtypeStruct((rows, cols), jnp.float32),
        interpret=interpret,
    )(ew2, m2)
    return out.reshape(E)


def _reduce_norm(parts, B):
    NW = parts.shape[0]
    R = NW // B
    aggr = parts[0:B]
    for k in range(1, R):
        aggr = aggr + parts[k * B:(k + 1) * B]
    mn = jnp.min(aggr)
    mx = jnp.max(aggr)
    return (aggr - mn) / (mx - mn)


def _make_update(B, N, NW, interpret=False):
    def body(parts_ref, thr_ref, h_ref):
        t = _reduce_norm(parts_ref[...], B)
        h_ref[...] = jax.nn.sigmoid(t - jnp.abs(thr_ref[...]))

    return pl.pallas_call(
        body,
        out_shape=jax.ShapeDtypeStruct((B, N), jnp.float32),
        interpret=interpret,
    )


def _make_final(B, N, NW, n_classes, interpret=False):
    def body(parts_ref, thr_ref, mask_ref, wfc_ref, bfc_ref, out_ref):
        t = _reduce_norm(parts_ref[...], B)
        h = jax.nn.sigmoid(t - jnp.abs(thr_ref[...]))
        pooled = jnp.sum(h * mask_ref[...], axis=1, keepdims=True)
        out_ref[...] = pooled * wfc_ref[...] + bfc_ref[...]

    return pl.pallas_call(
        body,
        out_shape=jax.ShapeDtypeStruct((B, n_classes), jnp.float32),
        interpret=interpret,
    )


def kernel(x, edge_index, edge_weight, edge_weight_multiplier,
           neuron_activation_threshold, W_fc, b_fc, sel_idx):
    N = neuron_activation_threshold.shape[0]
    B = x.shape[0] // N
    E = edge_weight.shape[0]
    S = sel_idx.shape[0]
    n_classes = W_fc.shape[0]
    NW = _NC * _NS

    h = x.reshape(B, N)
    src = edge_index[0]
    dst = edge_index[1]
    thr2 = neuron_activation_threshold.reshape(1, N)
    maskw = jnp.zeros((N,), jnp.float32).at[sel_idx].set(1.0 / S).reshape(1, N)
    wfc_row = W_fc.reshape(1, n_classes)
    bfc_row = b_fc.reshape(1, n_classes)

    effw = _compute_effw(edge_weight, edge_weight_multiplier)
    prop = _make_propagate(B, N, E)
    update = _make_update(B, N, NW)
    final = _make_final(B, N, NW, n_classes)

    for p in range(_NUM_PASSES):
        parts = prop(h, src, dst, effw)
        if p < _NUM_PASSES - 1:
            h = update(parts, thr2)
        else:
            out = final(parts, thr2, maskw, wfc_row, bfc_row)
    return out


# speedup vs baseline: 23.3482x; 1.5050x over previous
import functools

import jax
import jax.numpy as jnp
from jax import lax
from jax.experimental import pallas as pl
from jax.experimental.pallas import tpu as pltpu
from jax.experimental.pallas import tpu_sc as plsc

_NUM_PASSES = 3
_NC = 2
_NS = 16
_LANES = 16


def _pick_chunk(epw: int) -> int:
    for c in range(4096, 15, -16):
        if epw % c == 0 and (epw // c) % 2 == 0:
            return c
    return _LANES


def _make_propagate(B: int, N: int, E: int):
    NW = _NC * _NS
    R = NW // B
    EPW = E // R
    C = _pick_chunk(EPW)
    n_chunks = EPW // C

    mesh = plsc.VectorSubcoreMesh(
        core_axis_name="c", subcore_axis_name="s",
        num_cores=_NC, num_subcores=_NS)

    @functools.partial(
        pl.kernel,
        out_type=jax.ShapeDtypeStruct((NW, N), jnp.float32),
        mesh=mesh,
        scratch_types=[
            pltpu.VMEM((N,), jnp.float32),
            pltpu.VMEM((N,), jnp.float32),
            pltpu.VMEM((C,), jnp.int32),
            pltpu.VMEM((C,), jnp.int32),
            pltpu.VMEM((C,), jnp.float32),
            pltpu.VMEM((C,), jnp.float32),
            pltpu.SemaphoreType.DMA,
            pltpu.SemaphoreType.DMA,
            pltpu.SemaphoreType.DMA,
            pltpu.SemaphoreType.DMA,
        ],
        compiler_params=pltpu.CompilerParams(needs_layout_passes=False),
    )
    def prop(h_hbm, pair_hbm, w_hbm, out_hbm,
             h_v, aggr_v, pair0_v, pair1_v, w0_v, w1_v, sp0, sp1, sw0, sw1):
        cid = lax.axis_index("c")
        sid = lax.axis_index("s")
        wid = sid * _NC + cid
        b = wid % B
        r = wid // B
        base = r * EPW
        slots = ((pair0_v, w0_v, sp0, sw0), (pair1_v, w1_v, sp1, sw1))

        def issue(chunk_idx, slot):
            pv, wv, sp, sw = slots[slot]
            off = base + chunk_idx * C
            pltpu.async_copy(pair_hbm.at[pl.ds(off, C)], pv, sp)
            pltpu.async_copy(w_hbm.at[pl.ds(off, C)], wv, sw)

        issue(0, 0)
        pltpu.sync_copy(h_hbm.at[b], h_v)

        def zero_body(i, carry):
            aggr_v[pl.ds(i * _LANES, _LANES)] = jnp.zeros((_LANES,),
                                                          jnp.float32)
            return carry
        lax.fori_loop(0, N // _LANES, zero_body, 0, unroll=8)

        def process(slot):
            prow, wrow, sp, sw = slots[slot]
            pltpu.make_async_copy(pair_hbm.at[pl.ds(0, C)], prow, sp).wait()
            pltpu.make_async_copy(w_hbm.at[pl.ds(0, C)], wrow, sw).wait()

            def inner(i, icarry):
                sl = pl.ds(i * _LANES, _LANES)
                pr = prow[sl]
                wi = wrow[sl]
                si = pr & 0xFFFF
                di = lax.shift_right_logical(pr, 16)
                vals = plsc.load_gather(h_v, [si])
                plsc.addupdate_scatter(aggr_v, [di], vals * wi)
                return icarry
            lax.fori_loop(0, C // _LANES, inner, 0, unroll=8)

        def pair_body(k, carry):
            c0 = 2 * k
            issue(c0 + 1, 1)
            process(0)

            @pl.when(c0 + 2 < n_chunks)
            def _():
                issue(c0 + 2, 0)
            process(1)
            return carry
        lax.fori_loop(0, n_chunks // 2, pair_body, 0)

        pltpu.sync_copy(aggr_v, out_hbm.at[wid])

    return prop


def _prep_edges(src, dst, edge_weight, edge_weight_multiplier):
    E = edge_weight.shape[0]
    cols = 512
    rows = E // cols

    def body(src_ref, dst_ref, ew_ref, mult_ref, pair_ref, w_ref):
        pair_ref[...] = (dst_ref[...] << 16) | src_ref[...]
        w_ref[...] = ew_ref[...] * jax.nn.sigmoid(mult_ref[...])

    pair2, w2 = pl.pallas_call(
        body,
        out_shape=[jax.ShapeDtypeStruct((rows, cols), jnp.int32),
                   jax.ShapeDtypeStruct((rows, cols), jnp.float32)],
    )(src.reshape(rows, cols), dst.reshape(rows, cols),
      edge_weight.reshape(rows, cols),
      edge_weight_multiplier.reshape(rows, cols))
    return pair2.reshape(E), w2.reshape(E)


def _reduce_norm(parts, B):
    NW = parts.shape[0]
    R = NW // B
    aggr = parts[0:B]
    for k in range(1, R):
        aggr = aggr + parts[k * B:(k + 1) * B]
    mn = jnp.min(aggr)
    mx = jnp.max(aggr)
    return (aggr - mn) / (mx - mn)


def _make_update(B, N, NW):
    def body(parts_ref, thr_ref, h_ref):
        t = _reduce_norm(parts_ref[...], B)
        h_ref[...] = jax.nn.sigmoid(t - jnp.abs(thr_ref[...]))

    return pl.pallas_call(
        body,
        out_shape=jax.ShapeDtypeStruct((B, N), jnp.float32),
    )


def _make_final(B, N, NW, n_classes):
    def body(parts_ref, thr_ref, mask_ref, wfc_ref, bfc_ref, out_ref):
        t = _reduce_norm(parts_ref[...], B)
        h = jax.nn.sigmoid(t - jnp.abs(thr_ref[...]))
        pooled = jnp.sum(h * mask_ref[...], axis=1, keepdims=True)
        out_ref[...] = pooled * wfc_ref[...] + bfc_ref[...]

    return pl.pallas_call(
        body,
        out_shape=jax.ShapeDtypeStruct((B, n_classes), jnp.float32),
    )


def kernel(x, edge_index, edge_weight, edge_weight_multiplier,
           neuron_activation_threshold, W_fc, b_fc, sel_idx):
    N = neuron_activation_threshold.shape[0]
    B = x.shape[0] // N
    E = edge_weight.shape[0]
    S = sel_idx.shape[0]
    n_classes = W_fc.shape[0]
    NW = _NC * _NS

    h = x.reshape(B, N)
    src = edge_index[0]
    dst = edge_index[1]
    thr2 = neuron_activation_threshold.reshape(1, N)
    maskw = jnp.zeros((N,), jnp.float32).at[sel_idx].set(1.0 / S).reshape(1, N)
    wfc_row = W_fc.reshape(1, n_classes)
    bfc_row = b_fc.reshape(1, n_classes)

    pair, effw = _prep_edges(src, dst, edge_weight, edge_weight_multiplier)
    prop = _make_propagate(B, N, E)
    update = _make_update(B, N, NW)
    final = _make_final(B, N, NW, n_classes)

    for p in range(_NUM_PASSES):
        parts = prop(h, pair, effw)
        if p < _NUM_PASSES - 1:
            h = update(parts, thr2)
        else:
            out = final(parts, thr2, maskw, wfc_row, bfc_row)
    return out


# speedup vs baseline: 54.9071x; 2.3517x over previous
import functools

import jax
import jax.numpy as jnp
from jax import lax
from jax.experimental import pallas as pl
from jax.experimental.pallas import tpu as pltpu
from jax.experimental.pallas import tpu_sc as plsc

_NUM_PASSES = 3
_NC = 2
_NS = 16
_LANES = 16


def _pick_chunk(epw: int) -> int:
    for c in range(4096, 15, -16):
        if epw % c == 0 and (epw // c) % 2 == 0:
            return c
    return _LANES


def _make_propagate(B: int, N: int, E: int):
    NW = _NC * _NS
    R = NW // B
    EPW = E // R
    C = _pick_chunk(EPW)
    n_chunks = EPW // C

    mesh = plsc.VectorSubcoreMesh(
        core_axis_name="c", subcore_axis_name="s",
        num_cores=_NC, num_subcores=_NS)

    @functools.partial(
        pl.kernel,
        out_type=jax.ShapeDtypeStruct((NW, N), jnp.float32),
        mesh=mesh,
        scratch_types=[
            pltpu.VMEM((N,), jnp.float32),
            pltpu.VMEM((N,), jnp.float32),
            pltpu.VMEM((C,), jnp.int32),
            pltpu.VMEM((C,), jnp.int32),
            pltpu.VMEM((C,), jnp.float32),
            pltpu.VMEM((C,), jnp.float32),
            pltpu.SemaphoreType.DMA,
            pltpu.SemaphoreType.DMA,
            pltpu.SemaphoreType.DMA,
            pltpu.SemaphoreType.DMA,
        ],
        compiler_params=pltpu.CompilerParams(needs_layout_passes=False),
    )
    def prop(h_hbm, pair_hbm, w_hbm, out_hbm,
             h_v, aggr_v, pair0_v, pair1_v, w0_v, w1_v, sp0, sp1, sw0, sw1):
        cid = lax.axis_index("c")
        sid = lax.axis_index("s")
        wid = sid * _NC + cid
        b = wid % B
        r = wid // B
        base = r * EPW
        slots = ((pair0_v, w0_v, sp0, sw0), (pair1_v, w1_v, sp1, sw1))

        def issue(chunk_idx, slot):
            pv, wv, sp, sw = slots[slot]
            off = base + chunk_idx * C
            pltpu.async_copy(pair_hbm.at[pl.ds(off, C)], pv, sp)
            pltpu.async_copy(w_hbm.at[pl.ds(off, C)], wv, sw)

        issue(0, 0)
        pltpu.sync_copy(h_hbm.at[b], h_v)

        def zero_body(i, carry):
            aggr_v[pl.ds(i * _LANES, _LANES)] = jnp.zeros((_LANES,),
                                                          jnp.float32)
            return carry
        lax.fori_loop(0, N // _LANES, zero_body, 0, unroll=8)

        def process(slot):
            prow, wrow, sp, sw = slots[slot]
            pltpu.make_async_copy(pair_hbm.at[pl.ds(0, C)], prow, sp).wait()
            pltpu.make_async_copy(w_hbm.at[pl.ds(0, C)], wrow, sw).wait()

            @plsc.parallel_loop(0, C, _LANES, unroll=8)
            def _(off):
                sl = pl.ds(off, _LANES)
                pr = prow[sl]
                wi = wrow[sl]
                si = pr & 0xFFFF
                di = lax.shift_right_logical(pr, 16)
                vals = plsc.load_gather(h_v, [si])
                plsc.addupdate_scatter(aggr_v, [di], vals * wi)

        def pair_body(k, carry):
            c0 = 2 * k
            issue(c0 + 1, 1)
            process(0)

            @pl.when(c0 + 2 < n_chunks)
            def _():
                issue(c0 + 2, 0)
            process(1)
            return carry
        lax.fori_loop(0, n_chunks // 2, pair_body, 0)

        pltpu.sync_copy(aggr_v, out_hbm.at[wid])

    return prop


def _prep_edges(src, dst, edge_weight, edge_weight_multiplier):
    E = edge_weight.shape[0]
    cols = 512
    rows = E // cols

    def body(src_ref, dst_ref, ew_ref, mult_ref, pair_ref, w_ref):
        pair_ref[...] = (dst_ref[...] << 16) | src_ref[...]
        w_ref[...] = ew_ref[...] * jax.nn.sigmoid(mult_ref[...])

    pair2, w2 = pl.pallas_call(
        body,
        out_shape=[jax.ShapeDtypeStruct((rows, cols), jnp.int32),
                   jax.ShapeDtypeStruct((rows, cols), jnp.float32)],
    )(src.reshape(rows, cols), dst.reshape(rows, cols),
      edge_weight.reshape(rows, cols),
      edge_weight_multiplier.reshape(rows, cols))
    return pair2.reshape(E), w2.reshape(E)


def _reduce_norm(parts, B):
    NW = parts.shape[0]
    R = NW // B
    aggr = parts[0:B]
    for k in range(1, R):
        aggr = aggr + parts[k * B:(k + 1) * B]
    mn = jnp.min(aggr)
    mx = jnp.max(aggr)
    return (aggr - mn) / (mx - mn)


def _make_update(B, N, NW):
    def body(parts_ref, thr_ref, h_ref):
        t = _reduce_norm(parts_ref[...], B)
        h_ref[...] = jax.nn.sigmoid(t - jnp.abs(thr_ref[...]))

    return pl.pallas_call(
        body,
        out_shape=jax.ShapeDtypeStruct((B, N), jnp.float32),
    )


def _make_final(B, N, NW, n_classes):
    def body(parts_ref, thr_ref, mask_ref, wfc_ref, bfc_ref, out_ref):
        t = _reduce_norm(parts_ref[...], B)
        h = jax.nn.sigmoid(t - jnp.abs(thr_ref[...]))
        pooled = jnp.sum(h * mask_ref[...], axis=1, keepdims=True)
        out_ref[...] = pooled * wfc_ref[...] + bfc_ref[...]

    return pl.pallas_call(
        body,
        out_shape=jax.ShapeDtypeStruct((B, n_classes), jnp.float32),
    )


def kernel(x, edge_index, edge_weight, edge_weight_multiplier,
           neuron_activation_threshold, W_fc, b_fc, sel_idx):
    N = neuron_activation_threshold.shape[0]
    B = x.shape[0] // N
    E = edge_weight.shape[0]
    S = sel_idx.shape[0]
    n_classes = W_fc.shape[0]
    NW = _NC * _NS

    h = x.reshape(B, N)
    src = edge_index[0]
    dst = edge_index[1]
    thr2 = neuron_activation_threshold.reshape(1, N)
    maskw = jnp.zeros((N,), jnp.float32).at[sel_idx].set(1.0 / S).reshape(1, N)
    wfc_row = W_fc.reshape(1, n_classes)
    bfc_row = b_fc.reshape(1, n_classes)

    pair, effw = _prep_edges(src, dst, edge_weight, edge_weight_multiplier)
    prop = _make_propagate(B, N, E)
    update = _make_update(B, N, NW)
    final = _make_final(B, N, NW, n_classes)

    for p in range(_NUM_PASSES):
        parts = prop(h, pair, effw)
        if p < _NUM_PASSES - 1:
            h = update(parts, thr2)
        else:
            out = final(parts, thr2, maskw, wfc_row, bfc_row)
    return out


# speedup vs baseline: 58.9221x; 1.0731x over previous
import functools

import jax
import jax.numpy as jnp
from jax import lax
from jax.experimental import pallas as pl
from jax.experimental.pallas import tpu as pltpu
from jax.experimental.pallas import tpu_sc as plsc

_NUM_PASSES = 3
_NC = 2
_NS = 16
_LANES = 16


_DEPTH = 4


def _pick_chunk(epw: int, n: int) -> int:
    budget = (131071 - 2 * n) // (2 * _DEPTH)
    for c in range(budget - budget % 16, 15, -16):
        if epw % c == 0:
            return c
    return _LANES


def _make_propagate(B: int, N: int, E: int):
    NW = _NC * _NS
    R = NW // B
    EPW = E // R
    C = _pick_chunk(EPW, N)
    n_chunks = EPW // C
    D = _DEPTH
    full = n_chunks // D
    rem = n_chunks % D

    mesh = plsc.VectorSubcoreMesh(
        core_axis_name="c", subcore_axis_name="s",
        num_cores=_NC, num_subcores=_NS)

    @functools.partial(
        pl.kernel,
        out_type=jax.ShapeDtypeStruct((NW, N), jnp.float32),
        mesh=mesh,
        scratch_types=(
            [pltpu.VMEM((N,), jnp.float32),
             pltpu.VMEM((N,), jnp.float32)]
            + [pltpu.VMEM((2 * C,), jnp.int32)] * D
            + [pltpu.SemaphoreType.DMA] * D
        ),
        compiler_params=pltpu.CompilerParams(needs_layout_passes=False),
    )
    def prop(h_hbm, pw_hbm, out_hbm, h_v, aggr_v, *ring):
        bufs = ring[:D]
        sems = ring[D:]
        cid = lax.axis_index("c")
        sid = lax.axis_index("s")
        wid = sid * _NC + cid
        b = wid % B
        base = (wid // B) * EPW

        def issue(chunk_idx, slot):
            off = (base + chunk_idx * C) * 2
            pltpu.async_copy(pw_hbm.at[pl.ds(off, 2 * C)], bufs[slot],
                             sems[slot])

        for s in range(D - 1):
            issue(s, s)
        pltpu.sync_copy(h_hbm.at[b], h_v)

        def zero_body(i, carry):
            aggr_v[pl.ds(i * _LANES, _LANES)] = jnp.zeros((_LANES,),
                                                          jnp.float32)
            return carry
        lax.fori_loop(0, N // _LANES, zero_body, 0, unroll=8)

        def process(slot):
            buf = bufs[slot]
            pltpu.make_async_copy(pw_hbm.at[pl.ds(0, 2 * C)], buf,
                                  sems[slot]).wait()

            @plsc.parallel_loop(0, C, _LANES, unroll=8)
            def _(off):
                pr = buf[pl.ds(off, _LANES)]
                wi = plsc.bitcast(buf[pl.ds(C + off, _LANES)], jnp.float32)
                si = pr & 0xFFFF
                di = lax.shift_right_logical(pr, 16)
                vals = plsc.load_gather(h_v, [si])
                plsc.addupdate_scatter(aggr_v, [di], vals * wi)

        def ring_body(m, carry):
            c0 = m * D
            for ph in range(D):
                nxt = c0 + ph + (D - 1)

                @pl.when(nxt < n_chunks)
                def _():
                    issue(nxt, (ph + D - 1) % D)
                process(ph)
            return carry
        lax.fori_loop(0, full, ring_body, 0)
        for ph in range(rem):
            process(ph)

        pltpu.sync_copy(aggr_v, out_hbm.at[wid])

    return prop


def _prep_edges(src, dst, edge_weight, edge_weight_multiplier, C):
    E = edge_weight.shape[0]
    G = E // C
    BG = 1
    for cand in (20, 10, 8, 5, 4, 2):
        if G % cand == 0:
            BG = cand
            break

    def body(src_ref, dst_ref, ew_ref, mult_ref, out_ref):
        pair = (dst_ref[...] << 16) | src_ref[...]
        w = ew_ref[...] * jax.nn.sigmoid(mult_ref[...])
        out_ref[:, 0:1, :] = pair
        out_ref[:, 1:2, :] = lax.bitcast_convert_type(w, jnp.int32)

    spec_i = pl.BlockSpec((BG, 1, C), lambda i: (i, 0, 0))
    out = pl.pallas_call(
        body,
        grid=(G // BG,),
        in_specs=[spec_i] * 4,
        out_specs=pl.BlockSpec((BG, 2, C), lambda i: (i, 0, 0)),
        out_shape=jax.ShapeDtypeStruct((G, 2, C), jnp.int32),
    )(src.reshape(G, 1, C), dst.reshape(G, 1, C),
      edge_weight.reshape(G, 1, C),
      edge_weight_multiplier.reshape(G, 1, C))
    return out.reshape(2 * E)


def _reduce_norm(parts, B):
    NW = parts.shape[0]
    R = NW // B
    aggr = parts[0:B]
    for k in range(1, R):
        aggr = aggr + parts[k * B:(k + 1) * B]
    mn = jnp.min(aggr)
    mx = jnp.max(aggr)
    return (aggr - mn) / (mx - mn)


def _make_update(B, N, NW):
    def body(parts_ref, thr_ref, h_ref):
        t = _reduce_norm(parts_ref[...], B)
        h_ref[...] = jax.nn.sigmoid(t - jnp.abs(thr_ref[...]))

    return pl.pallas_call(
        body,
        out_shape=jax.ShapeDtypeStruct((B, N), jnp.float32),
    )


def _make_final(B, N, NW, n_classes):
    def body(parts_ref, thr_ref, mask_ref, wfc_ref, bfc_ref, out_ref):
        t = _reduce_norm(parts_ref[...], B)
        h = jax.nn.sigmoid(t - jnp.abs(thr_ref[...]))
        pooled = jnp.sum(h * mask_ref[...], axis=1, keepdims=True)
        out_ref[...] = pooled * wfc_ref[...] + bfc_ref[...]

    return pl.pallas_call(
        body,
        out_shape=jax.ShapeDtypeStruct((B, n_classes), jnp.float32),
    )


def kernel(x, edge_index, edge_weight, edge_weight_multiplier,
           neuron_activation_threshold, W_fc, b_fc, sel_idx):
    N = neuron_activation_threshold.shape[0]
    B = x.shape[0] // N
    E = edge_weight.shape[0]
    S = sel_idx.shape[0]
    n_classes = W_fc.shape[0]
    NW = _NC * _NS

    h = x.reshape(B, N)
    src = edge_index[0]
    dst = edge_index[1]
    thr2 = neuron_activation_threshold.reshape(1, N)
    maskw = jnp.zeros((N,), jnp.float32).at[sel_idx].set(1.0 / S).reshape(1, N)
    wfc_row = W_fc.reshape(1, n_classes)
    bfc_row = b_fc.reshape(1, n_classes)

    C = _pick_chunk(E // (NW // B), N)
    pairw = _prep_edges(src, dst, edge_weight, edge_weight_multiplier, C)
    prop = _make_propagate(B, N, E)
    update = _make_update(B, N, NW)
    final = _make_final(B, N, NW, n_classes)

    for p in range(_NUM_PASSES):
        parts = prop(h, pairw)
        if p < _NUM_PASSES - 1:
            h = update(parts, thr2)
        else:
            out = final(parts, thr2, maskw, wfc_row, bfc_row)
    return out
